# baseline (speedup 1.0000x reference)
import jax
import jax.numpy as jnp
from jax import lax
from jax.experimental import pallas as pl
from jax.experimental.pallas import tpu as pltpu
from jax.experimental.pallas import tpu_sc as plsc

N_FEAT = 1_000_000
N_DIM = 64
BATCH = 16384

_NC = 2
_NS = 16
_NW = _NC * _NS
_B_PER_W = BATCH // _NW
_CHUNK = 128
_NCHUNK = _B_PER_W // _CHUNK
_L = 16


def _gather_body(idx_hbm, vw2_hbm, bias_hbm, bias_out, vect_out_t,
                 idx_v, idxh_v, half_v, buf2, out_buf, bias_v, sem, bsem):
    wid = lax.axis_index("s") * _NC + lax.axis_index("c")
    base = wid * _B_PER_W
    pltpu.sync_copy(idx_hbm.at[pl.ds(base, _B_PER_W)], idx_v)

    def split_body(q, carry):
        v = idx_v[pl.ds(q * _L, _L)]
        idxh_v[pl.ds(q * _L, _L)] = v >> 1
        half_v[pl.ds(q * _L, _L)] = (v & 1) * N_DIM
        return carry

    lax.fori_loop(0, _B_PER_W // _L, split_body, 0)

    bias_copies = [
        pltpu.async_copy(bias_hbm.at[idx_v.at[pl.ds(j * _CHUNK, _CHUNK)]],
                         bias_v.at[pl.ds(j * _CHUNK, _CHUNK)], bsem)
        for j in range(_NCHUNK)
    ]
    vect_copies = [
        pltpu.async_copy(vw2_hbm.at[idxh_v.at[pl.ds(j * _CHUNK, _CHUNK)]],
                         buf2.at[pl.ds(j * _CHUNK, _CHUNK)], sem)
        for j in range(_NCHUNK)
    ]

    lanes = lax.iota(jnp.int32, _L)

    for c in range(_NCHUNK):
        vect_copies[c].wait()

        def extract_body(j, carry):
            rows16 = j * _L + lanes
            halves = half_v[pl.ds(j * _L, _L)]
            for f in range(N_DIM):
                vals = plsc.load_gather(buf2, [rows16, halves + f])
                plsc.store_scatter(out_buf, [jnp.full((_L,), f, jnp.int32),
                                             rows16], vals)
            return carry

        lo = c * (_CHUNK // _L)
        lax.fori_loop(lo, lo + _CHUNK // _L, extract_body, 0)

    for c in bias_copies:
        c.wait()

    pltpu.sync_copy(out_buf, vect_out_t.at[:, pl.ds(base, _B_PER_W)])
    pltpu.sync_copy(bias_v, bias_out.at[pl.ds(base, _B_PER_W)])


def kernel(index, vect_weight, bias_weight):
    idx = index.astype(jnp.int32)
    vw2 = vect_weight.reshape(N_FEAT // 2, 2 * N_DIM)
    bias_flat = bias_weight.reshape(N_FEAT)
    mesh = plsc.VectorSubcoreMesh(core_axis_name="c", subcore_axis_name="s")
    k = pl.kernel(
        _gather_body,
        mesh=mesh,
        out_type=(
            jax.ShapeDtypeStruct((BATCH,), jnp.float32),
            jax.ShapeDtypeStruct((N_DIM, BATCH), jnp.float32),
        ),
        scratch_types=[
            pltpu.VMEM((_B_PER_W,), jnp.int32),
            pltpu.VMEM((_B_PER_W,), jnp.int32),
            pltpu.VMEM((_B_PER_W,), jnp.int32),
            pltpu.VMEM((_B_PER_W, 2 * N_DIM), jnp.float32),
            pltpu.VMEM((N_DIM, _B_PER_W), jnp.float32),
            pltpu.VMEM((_B_PER_W,), jnp.float32),
            pltpu.SemaphoreType.DMA,
            pltpu.SemaphoreType.DMA,
        ],
        compiler_params=pltpu.CompilerParams(use_tc_tiling_on_sc=True,
                                             needs_layout_passes=False),
    )
    bias_out, vect_out_t = k(idx, vw2, bias_flat)
    return bias_out, vect_out_t.T


# speedup vs baseline: 1.0189x; 1.0189x over previous
import jax
import jax.numpy as jnp
from jax import lax
from jax.experimental import pallas as pl
from jax.experimental.pallas import tpu as pltpu
from jax.experimental.pallas import tpu_sc as plsc

N_FEAT = 1_000_000
N_DIM = 64
BATCH = 16384

_NC = 2
_NS = 16
_NW = _NC * _NS
_CHUNK = 128
_B_PER_W = BATCH // _NW
_NCHUNK = _B_PER_W // _CHUNK


def _gather_body(idx_hbm, vect_hbm, bias_hbm, bias_out, vect_out,
                 idx_v, rows_v, bias_v, sem):
    wid = lax.axis_index("s") * _NC + lax.axis_index("c")
    base = wid * _B_PER_W
    pltpu.sync_copy(idx_hbm.at[wid], idx_v)
    copies = []
    for j in range(_NCHUNK):
        copies.append(pltpu.async_copy(
            vect_hbm.at[idx_v.at[j]],
            rows_v.at[pl.ds(j * _CHUNK, _CHUNK)], sem))
        copies.append(pltpu.async_copy(
            bias_hbm.at[idx_v.at[j]],
            bias_v.at[pl.ds(j * _CHUNK, _CHUNK)], sem))
    for c in copies:
        c.wait()
    pltpu.sync_copy(rows_v, vect_out.at[pl.ds(base, _B_PER_W)])
    pltpu.sync_copy(bias_v, bias_out.at[pl.ds(base, _B_PER_W)])


def kernel(index, vect_weight, bias_weight):
    idx = index.astype(jnp.int32).reshape(_NW, _NCHUNK, _CHUNK)
    bias_flat = bias_weight.reshape(N_FEAT)
    mesh = plsc.VectorSubcoreMesh(core_axis_name="c", subcore_axis_name="s")
    k = pl.kernel(
        _gather_body,
        mesh=mesh,
        out_type=(
            jax.ShapeDtypeStruct((BATCH,), jnp.float32),
            jax.ShapeDtypeStruct((BATCH, N_DIM), jnp.float32),
        ),
        scratch_types=[
            pltpu.VMEM((_NCHUNK, _CHUNK), jnp.int32),
            pltpu.VMEM((_B_PER_W, N_DIM), jnp.float32),
            pltpu.VMEM((_B_PER_W,), jnp.float32),
            pltpu.SemaphoreType.DMA,
        ],
        compiler_params=pltpu.CompilerParams(use_tc_tiling_on_sc=False),
    )
    bias_out, vect_out = k(idx, vect_weight, bias_flat)
    return bias_out, vect_out


# speedup vs baseline: 1.9588x; 1.9224x over previous
import jax
import jax.numpy as jnp
from jax import lax
from jax.experimental import pallas as pl
from jax.experimental.pallas import tpu as pltpu
from jax.experimental.pallas import tpu_sc as plsc

N_FEAT = 1_000_000
N_DIM = 64
BATCH = 16384

_NC = 2
_NS = 16
_NW = _NC * _NS
_B_PER_W = BATCH // _NW
_L = 16

_TC_FULL = N_FEAT // 128
_TAIL_LO = _TC_FULL * 128
_N_TAIL = N_FEAT - _TAIL_LO
_TC_BASE = _TC_FULL // _NW
_TC_EXTRA = _TC_FULL % _NW
_CH = 4
_NCHUNK = (_TC_BASE + 1 + _CH - 1) // _CH
_LAST_C0 = _TC_FULL - _CH
_W_ROWS = _CH * 128
_PAD_ROWS = BATCH + _NW


def _body(idx_hbm, vwt_hbm, tail_hbm, bias_hbm, bias_out, out_pad,
          idx_v, jlist, cj, tj, stage, tail_s, outg, jg, bias_v,
          sem, ssem, bsem):
    wid = lax.axis_index("s") * _NC + lax.axis_index("c")
    lanes = lax.iota(jnp.int32, _L)

    pltpu.sync_copy(idx_hbm, idx_v)
    pltpu.sync_copy(tail_hbm, tail_s)

    bias_copies = [
        pltpu.async_copy(
            bias_hbm.at[idx_v.at[pl.ds(wid * _B_PER_W + j * 128, 128)]],
            bias_v.at[pl.ds(j * 128, 128)], bsem)
        for j in range(_B_PER_W // 128)
    ]

    base_tc = wid * _TC_BASE + jnp.minimum(wid, _TC_EXTRA)
    ntc = _TC_BASE + jnp.where(wid < _TC_EXTRA, 1, 0)
    slab_lo = base_tc * 128
    slab_len = (ntc * 128).astype(jnp.uint32)

    def scan1(g, kw):
        iv = idx_v[pl.ds(g * _L, _L)]
        m = (iv - slab_lo).astype(jnp.uint32) < slab_len
        mi = jnp.where(m, 1, 0).astype(jnp.int32)
        pos = kw + plsc.cumsum(mi) - 1
        plsc.store_scatter(jlist, [pos], g * _L + lanes, mask=m)
        cnt = plsc.all_reduce_population_count(m)
        return kw + lax.reduce_max(cnt, (0,))

    kw = lax.fori_loop(0, BATCH // _L, scan1, jnp.int32(0))

    def chunk_body(i, carry):
        c0 = jnp.minimum(base_tc + i * _CH, _LAST_C0)
        cps = [
            pltpu.async_copy(
                vwt_hbm.at[pl.ds(tr * 8, 8), pl.ds(c0 * 128, _W_ROWS)],
                stage.at[tr], sem)
            for tr in range(8)
        ]
        for c in cps:
            c.wait()

        def scan2(g, kc):
            valid = (g * _L + lanes) < kw
            jv = jlist[pl.ds(g * _L, _L)]
            js = jnp.where(valid, jv, 0)
            rv = plsc.load_gather(idx_v, [js])
            m = valid & ((rv - c0 * 128).astype(jnp.uint32)
                         < jnp.uint32(_W_ROWS))
            mi = jnp.where(m, 1, 0).astype(jnp.int32)
            pos = kc + plsc.cumsum(mi) - 1
            plsc.store_scatter(cj, [pos], js, mask=m)
            cnt = plsc.all_reduce_population_count(m)
            return kc + lax.reduce_max(cnt, (0,))

        kc = lax.fori_loop(0, (kw + _L - 1) >> 4, scan2, jnp.int32(0))

        def grp(g2, carry2):
            for t in range(_L):
                it = g2 * _L + t
                vm = jnp.full((_L,), it, jnp.int32) < kc
                jx = plsc.load_gather(cj, [jnp.full((_L,), it, jnp.int32)])
                js2 = jnp.where(vm, jx, 0)
                rv = plsc.load_gather(idx_v, [js2])
                x = jnp.where(vm, rv - c0 * 128, 0)
                jdst = jnp.where(vm, js2, BATCH + wid)
                for q in range(N_DIM // _L):
                    w = q * _L + lanes
                    vals = plsc.load_gather(stage, [w >> 3, w & 7, x])
                    outg[t, pl.ds(q * _L, _L)] = vals
                plsc.store_scatter(jg, [jnp.full((_L,), t, jnp.int32)],
                                   jdst, mask=(lanes == 0))
            pltpu.async_copy(outg, out_pad.at[jg], ssem).wait()
            return carry2

        lax.fori_loop(0, (kc + _L - 1) >> 4, grp, jnp.int32(0))
        return carry

    lax.fori_loop(0, _NCHUNK, chunk_body, jnp.int32(0))

    def scant(g, kt):
        iv = idx_v[pl.ds(wid * _B_PER_W + g * _L, _L)]
        m = iv >= _TAIL_LO
        mi = jnp.where(m, 1, 0).astype(jnp.int32)
        pos = kt + plsc.cumsum(mi) - 1
        plsc.store_scatter(tj, [pos], wid * _B_PER_W + g * _L + lanes,
                           mask=m)
        cnt = plsc.all_reduce_population_count(m)
        return kt + lax.reduce_max(cnt, (0,))

    kt = lax.fori_loop(0, _B_PER_W // _L, scant, jnp.int32(0))

    def grpt(g2, carry2):
        for t in range(_L):
            it = g2 * _L + t
            vm = jnp.full((_L,), it, jnp.int32) < kt
            jx = plsc.load_gather(tj, [jnp.full((_L,), it, jnp.int32)])
            js2 = jnp.where(vm, jx, 0)
            rv = plsc.load_gather(idx_v, [js2])
            rt = jnp.where(vm, rv - _TAIL_LO, 0)
            jdst = jnp.where(vm, js2, BATCH + wid)
            for q in range(N_DIM // _L):
                w = q * _L + lanes
                vals = plsc.load_gather(tail_s, [rt, w])
                outg[t, pl.ds(q * _L, _L)] = vals
            plsc.store_scatter(jg, [jnp.full((_L,), t, jnp.int32)],
                               jdst, mask=(lanes == 0))
        pltpu.async_copy(outg, out_pad.at[jg], ssem).wait()
        return carry2

    lax.fori_loop(0, (kt + _L - 1) >> 4, grpt, jnp.int32(0))

    for c in bias_copies:
        c.wait()
    pltpu.sync_copy(bias_v, bias_out.at[pl.ds(wid * _B_PER_W, _B_PER_W)])


def kernel(index, vect_weight, bias_weight):
    idx = index.astype(jnp.int32)
    vw_t = vect_weight.T
    tail = lax.slice(vect_weight, (_TAIL_LO, 0), (N_FEAT, N_DIM))
    bias_flat = bias_weight.reshape(N_FEAT)
    mesh = plsc.VectorSubcoreMesh(core_axis_name="c", subcore_axis_name="s")
    k = pl.kernel(
        _body,
        mesh=mesh,
        out_type=(
            jax.ShapeDtypeStruct((BATCH,), jnp.float32),
            jax.ShapeDtypeStruct((_PAD_ROWS, 128), jnp.float32),
        ),
        scratch_types=[
            pltpu.VMEM((BATCH,), jnp.int32),
            pltpu.VMEM((BATCH,), jnp.int32),
            pltpu.VMEM((BATCH,), jnp.int32),
            pltpu.VMEM((_B_PER_W,), jnp.int32),
            pltpu.VMEM((8, 8, _W_ROWS), jnp.float32),
            pltpu.VMEM((_N_TAIL, N_DIM), jnp.float32),
            pltpu.VMEM((_L, 128), jnp.float32),
            pltpu.VMEM((_L,), jnp.int32),
            pltpu.VMEM((_B_PER_W,), jnp.float32),
            pltpu.SemaphoreType.DMA,
            pltpu.SemaphoreType.DMA,
            pltpu.SemaphoreType.DMA,
        ],
        compiler_params=pltpu.CompilerParams(use_tc_tiling_on_sc=True,
                                             needs_layout_passes=False),
    )
    bias_out, out_pad = k(idx, vw_t, tail, bias_flat)
    vect_out = lax.slice(out_pad, (0, 0), (BATCH, N_DIM))
    return bias_out, vect_out


# speedup vs baseline: 2.3746x; 1.2123x over previous
import jax
import jax.numpy as jnp
from jax import lax
from jax.experimental import pallas as pl
from jax.experimental.pallas import tpu as pltpu
from jax.experimental.pallas import tpu_sc as plsc

N_FEAT = 1_000_000
N_DIM = 64
BATCH = 16384

_NC = 2
_NS = 16
_NW = _NC * _NS
_B_PER_W = BATCH // _NW
_L = 16

_TC_FULL = N_FEAT // 128
_TAIL_LO = _TC_FULL * 128
_N_TAIL = N_FEAT - _TAIL_LO
_TC_BASE = _TC_FULL // _NW
_TC_EXTRA = _TC_FULL % _NW
_CH = 8
_NCHUNK = (_TC_BASE + 1 + _CH - 1) // _CH
_LAST_C0 = _TC_FULL - _CH
_W_ROWS = _CH * 128
_PAD_ROWS = BATCH + _NW


def _body(idx_hbm, vwt_hbm, tail_hbm, bias_hbm, bias_out, out_pad,
          idx_v, jlist, cj, tj, stage, tail_s, outg, jg, bias_v,
          sem, ssem, bsem):
    wid = lax.axis_index("s") * _NC + lax.axis_index("c")
    lanes = lax.iota(jnp.int32, _L)

    pltpu.sync_copy(idx_hbm, idx_v)
    pltpu.sync_copy(tail_hbm, tail_s)

    bias_copies = [
        pltpu.async_copy(
            bias_hbm.at[idx_v.at[pl.ds(wid * _B_PER_W + j * 128, 128)]],
            bias_v.at[pl.ds(j * 128, 128)], bsem)
        for j in range(_B_PER_W // 128)
    ]

    base_tc = wid * _TC_BASE + jnp.minimum(wid, _TC_EXTRA)
    ntc = _TC_BASE + jnp.where(wid < _TC_EXTRA, 1, 0)
    slab_lo = base_tc * 128
    slab_len = (ntc * 128).astype(jnp.uint32)

    def scan1(g, kw):
        iv = idx_v[pl.ds(g * _L, _L)]
        m = (iv - slab_lo).astype(jnp.uint32) < slab_len
        mi = jnp.where(m, 1, 0).astype(jnp.int32)
        pos = kw + plsc.cumsum(mi) - 1
        plsc.store_scatter(jlist, [pos], g * _L + lanes, mask=m)
        cnt = plsc.all_reduce_population_count(m)
        return kw + lax.reduce_max(cnt, (0,))

    kw = lax.fori_loop(0, BATCH // _L, scan1, jnp.int32(0))

    def chunk_body(i, carry):
        c0 = jnp.minimum(base_tc + i * _CH, _LAST_C0)
        cps = [
            pltpu.async_copy(
                vwt_hbm.at[pl.ds(tr * 8, 8), pl.ds(c0 * 128, _W_ROWS)],
                stage.at[tr], sem)
            for tr in range(8)
        ]
        for c in cps:
            c.wait()

        def scan2(g, kc):
            valid = (g * _L + lanes) < kw
            jv = jlist[pl.ds(g * _L, _L)]
            js = jnp.where(valid, jv, 0)
            rv = plsc.load_gather(idx_v, [js])
            m = valid & ((rv - c0 * 128).astype(jnp.uint32)
                         < jnp.uint32(_W_ROWS))
            mi = jnp.where(m, 1, 0).astype(jnp.int32)
            pos = kc + plsc.cumsum(mi) - 1
            plsc.store_scatter(cj, [pos], js, mask=m)
            cnt = plsc.all_reduce_population_count(m)
            return kc + lax.reduce_max(cnt, (0,))

        kc = lax.fori_loop(0, (kw + _L - 1) >> 4, scan2, jnp.int32(0))

        def grp(g2, carry2):
            for t in range(_L):
                it = g2 * _L + t
                vm = jnp.full((_L,), it, jnp.int32) < kc
                jx = plsc.load_gather(cj, [jnp.full((_L,), it, jnp.int32)])
                js2 = jnp.where(vm, jx, 0)
                rv = plsc.load_gather(idx_v, [js2])
                x = jnp.where(vm, rv - c0 * 128, 0)
                jdst = jnp.where(vm, js2, BATCH + wid)
                for q in range(N_DIM // _L):
                    w = q * _L + lanes
                    vals = plsc.load_gather(stage, [w >> 3, w & 7, x])
                    outg[t, pl.ds(q * _L, _L)] = vals
                plsc.store_scatter(jg, [jnp.full((_L,), t, jnp.int32)],
                                   jdst, mask=(lanes == 0))
            pltpu.async_copy(outg, out_pad.at[jg], ssem).wait()
            return carry2

        lax.fori_loop(0, (kc + _L - 1) >> 4, grp, jnp.int32(0))
        return carry

    lax.fori_loop(0, _NCHUNK, chunk_body, jnp.int32(0))

    def scant(g, kt):
        iv = idx_v[pl.ds(wid * _B_PER_W + g * _L, _L)]
        m = iv >= _TAIL_LO
        mi = jnp.where(m, 1, 0).astype(jnp.int32)
        pos = kt + plsc.cumsum(mi) - 1
        plsc.store_scatter(tj, [pos], wid * _B_PER_W + g * _L + lanes,
                           mask=m)
        cnt = plsc.all_reduce_population_count(m)
        return kt + lax.reduce_max(cnt, (0,))

    kt = lax.fori_loop(0, _B_PER_W // _L, scant, jnp.int32(0))

    def grpt(g2, carry2):
        for t in range(_L):
            it = g2 * _L + t
            vm = jnp.full((_L,), it, jnp.int32) < kt
            jx = plsc.load_gather(tj, [jnp.full((_L,), it, jnp.int32)])
            js2 = jnp.where(vm, jx, 0)
            rv = plsc.load_gather(idx_v, [js2])
            rt = jnp.where(vm, rv - _TAIL_LO, 0)
            jdst = jnp.where(vm, js2, BATCH + wid)
            for q in range(N_DIM // _L):
                w = q * _L + lanes
                vals = plsc.load_gather(tail_s, [rt, w])
                outg[t, pl.ds(q * _L, _L)] = vals
            plsc.store_scatter(jg, [jnp.full((_L,), t, jnp.int32)],
                               jdst, mask=(lanes == 0))
        pltpu.async_copy(outg, out_pad.at[jg], ssem).wait()
        return carry2

    lax.fori_loop(0, (kt + _L - 1) >> 4, grpt, jnp.int32(0))

    for c in bias_copies:
        c.wait()
    pltpu.sync_copy(bias_v, bias_out.at[pl.ds(wid * _B_PER_W, _B_PER_W)])


def kernel(index, vect_weight, bias_weight):
    idx = index.astype(jnp.int32)
    vw_t = vect_weight.T
    tail = lax.slice(vect_weight, (_TAIL_LO, 0), (N_FEAT, N_DIM))
    bias_flat = bias_weight.reshape(N_FEAT)
    mesh = plsc.VectorSubcoreMesh(core_axis_name="c", subcore_axis_name="s")
    k = pl.kernel(
        _body,
        mesh=mesh,
        out_type=(
            jax.ShapeDtypeStruct((BATCH,), jnp.float32),
            jax.ShapeDtypeStruct((_PAD_ROWS, 128), jnp.float32),
        ),
        scratch_types=[
            pltpu.VMEM((BATCH,), jnp.int32),
            pltpu.VMEM((BATCH,), jnp.int32),
            pltpu.VMEM((BATCH,), jnp.int32),
            pltpu.VMEM((_B_PER_W,), jnp.int32),
            pltpu.VMEM((8, 8, _W_ROWS), jnp.float32),
            pltpu.VMEM((_N_TAIL, N_DIM), jnp.float32),
            pltpu.VMEM((_L, 128), jnp.float32),
            pltpu.VMEM((_L,), jnp.int32),
            pltpu.VMEM((_B_PER_W,), jnp.float32),
            pltpu.SemaphoreType.DMA,
            pltpu.SemaphoreType.DMA,
            pltpu.SemaphoreType.DMA,
        ],
        compiler_params=pltpu.CompilerParams(use_tc_tiling_on_sc=True,
                                             needs_layout_passes=False),
    )
    bias_out, out_pad = k(idx, vw_t, tail, bias_flat)
    vect_out = lax.slice(out_pad, (0, 0), (BATCH, N_DIM))
    return bias_out, vect_out


# speedup vs baseline: 2.6103x; 1.0993x over previous
import jax
import jax.numpy as jnp
from jax import lax
from jax.experimental import pallas as pl
from jax.experimental.pallas import tpu as pltpu
from jax.experimental.pallas import tpu_sc as plsc

N_FEAT = 1_000_000
N_DIM = 64
BATCH = 16384

_NC = 2
_NS = 16
_NW = _NC * _NS
_B_PER_W = BATCH // _NW
_L = 16

_TC_FULL = N_FEAT // 128
_TAIL_LO = _TC_FULL * 128
_N_TAIL = N_FEAT - _TAIL_LO
_TC_BASE = _TC_FULL // _NW
_TC_EXTRA = _TC_FULL % _NW
_CH = 4
_NCHUNK = (_TC_BASE + 1 + _CH - 1) // _CH
_LAST_C0 = _TC_FULL - _CH
_W_ROWS = _CH * 128
_PAD_ROWS = BATCH + _NW


def _body(idx_hbm, vwt_hbm, tail_hbm, bias_hbm, bias_out, out_pad,
          idx_v, jlist, cj, tj, stage0, stage1, tail_s, outg, jg, bias_v,
          sem, semb, ssem, bsem):
    wid = lax.axis_index("s") * _NC + lax.axis_index("c")
    lanes = lax.iota(jnp.int32, _L)

    pltpu.sync_copy(idx_hbm, idx_v)
    pltpu.sync_copy(tail_hbm, tail_s)

    bias_copies = [
        pltpu.async_copy(
            bias_hbm.at[idx_v.at[pl.ds(wid * _B_PER_W + j * 128, 128)]],
            bias_v.at[pl.ds(j * 128, 128)], bsem)
        for j in range(_B_PER_W // 128)
    ]

    base_tc = wid * _TC_BASE + jnp.minimum(wid, _TC_EXTRA)
    ntc = _TC_BASE + jnp.where(wid < _TC_EXTRA, 1, 0)
    slab_lo = base_tc * 128
    slab_len = (ntc * 128).astype(jnp.uint32)

    def scan1(g, kw):
        iv = idx_v[pl.ds(g * _L, _L)]
        m = (iv - slab_lo).astype(jnp.uint32) < slab_len
        mi = jnp.where(m, 1, 0).astype(jnp.int32)
        pos = kw + plsc.cumsum(mi) - 1
        plsc.store_scatter(jlist, [pos], g * _L + lanes, mask=m)
        cnt = plsc.all_reduce_population_count(m)
        return kw + lax.reduce_max(cnt, (0,))

    kw = lax.fori_loop(0, BATCH // _L, scan1, jnp.int32(0))

    def _c0(i):
        return jnp.minimum(base_tc + i * _CH, _LAST_C0)

    def _issue(i, st, sm):
        c0 = _c0(i)
        for tr in range(8):
            pltpu.async_copy(
                vwt_hbm.at[pl.ds(tr * 8, 8), pl.ds(c0 * 128, _W_ROWS)],
                st.at[tr], sm)

    def _drain(i, st, sm):
        c0 = _c0(i)
        for tr in range(8):
            pltpu.make_async_copy(
                vwt_hbm.at[pl.ds(tr * 8, 8), pl.ds(c0 * 128, _W_ROWS)],
                st.at[tr], sm).wait()

    def _extract(c0, st):
        def scan2(g, kc):
            valid = (g * _L + lanes) < kw
            jv = jlist[pl.ds(g * _L, _L)]
            js = jnp.where(valid, jv, 0)
            rv = plsc.load_gather(idx_v, [js])
            m = valid & ((rv - c0 * 128).astype(jnp.uint32)
                         < jnp.uint32(_W_ROWS))
            mi = jnp.where(m, 1, 0).astype(jnp.int32)
            pos = kc + plsc.cumsum(mi) - 1
            plsc.store_scatter(cj, [pos], js, mask=m)
            cnt = plsc.all_reduce_population_count(m)
            return kc + lax.reduce_max(cnt, (0,))

        kc = lax.fori_loop(0, (kw + _L - 1) >> 4, scan2, jnp.int32(0))

        def grp(g2, carry2):
            for t in range(_L):
                it = g2 * _L + t
                vm = jnp.full((_L,), it, jnp.int32) < kc
                jx = plsc.load_gather(cj, [jnp.full((_L,), it, jnp.int32)])
                js2 = jnp.where(vm, jx, 0)
                rv = plsc.load_gather(idx_v, [js2])
                x = jnp.where(vm, rv - c0 * 128, 0)
                jdst = jnp.where(vm, js2, BATCH + wid)
                for q in range(N_DIM // _L):
                    w = q * _L + lanes
                    vals = plsc.load_gather(st, [w >> 3, w & 7, x])
                    outg[t, pl.ds(q * _L, _L)] = vals
                plsc.store_scatter(jg, [jnp.full((_L,), t, jnp.int32)],
                                   jdst, mask=(lanes == 0))
            pltpu.async_copy(outg, out_pad.at[jg], ssem).wait()
            return carry2

        lax.fori_loop(0, (kc + _L - 1) >> 4, grp, jnp.int32(0))

    _issue(0, stage0, sem)
    _issue(1, stage1, semb)

    def chunk_body(j, carry):
        i0 = 2 * j
        _drain(i0, stage0, sem)
        _extract(_c0(i0), stage0)
        _issue(i0 + 2, stage0, sem)
        i1 = 2 * j + 1
        _drain(i1, stage1, semb)
        _extract(_c0(i1), stage1)
        _issue(i1 + 2, stage1, semb)
        return carry

    lax.fori_loop(0, _NCHUNK // 2, chunk_body, jnp.int32(0))
    if _NCHUNK % 2:
        _drain(_NCHUNK - 1, stage0, sem)
        _extract(_c0(_NCHUNK - 1), stage0)
        _drain(_NCHUNK, stage1, semb)
    else:
        _drain(_NCHUNK, stage0, sem)
        _drain(_NCHUNK + 1, stage1, semb)

    def scant(g, kt):
        iv = idx_v[pl.ds(wid * _B_PER_W + g * _L, _L)]
        m = iv >= _TAIL_LO
        mi = jnp.where(m, 1, 0).astype(jnp.int32)
        pos = kt + plsc.cumsum(mi) - 1
        plsc.store_scatter(tj, [pos], wid * _B_PER_W + g * _L + lanes,
                           mask=m)
        cnt = plsc.all_reduce_population_count(m)
        return kt + lax.reduce_max(cnt, (0,))

    kt = lax.fori_loop(0, _B_PER_W // _L, scant, jnp.int32(0))

    def grpt(g2, carry2):
        for t in range(_L):
            it = g2 * _L + t
            vm = jnp.full((_L,), it, jnp.int32) < kt
            jx = plsc.load_gather(tj, [jnp.full((_L,), it, jnp.int32)])
            js2 = jnp.where(vm, jx, 0)
            rv = plsc.load_gather(idx_v, [js2])
            rt = jnp.where(vm, rv - _TAIL_LO, 0)
            jdst = jnp.where(vm, js2, BATCH + wid)
            for q in range(N_DIM // _L):
                w = q * _L + lanes
                vals = plsc.load_gather(tail_s, [rt, w])
                outg[t, pl.ds(q * _L, _L)] = vals
            plsc.store_scatter(jg, [jnp.full((_L,), t, jnp.int32)],
                               jdst, mask=(lanes == 0))
        pltpu.async_copy(outg, out_pad.at[jg], ssem).wait()
        return carry2

    lax.fori_loop(0, (kt + _L - 1) >> 4, grpt, jnp.int32(0))

    for c in bias_copies:
        c.wait()
    pltpu.sync_copy(bias_v, bias_out.at[pl.ds(wid * _B_PER_W, _B_PER_W)])


def kernel(index, vect_weight, bias_weight):
    idx = index.astype(jnp.int32)
    vw_t = vect_weight.T
    tail = lax.slice(vect_weight, (_TAIL_LO, 0), (N_FEAT, N_DIM))
    bias_flat = bias_weight.reshape(N_FEAT)
    mesh = plsc.VectorSubcoreMesh(core_axis_name="c", subcore_axis_name="s")
    k = pl.kernel(
        _body,
        mesh=mesh,
        out_type=(
            jax.ShapeDtypeStruct((BATCH,), jnp.float32),
            jax.ShapeDtypeStruct((_PAD_ROWS, 128), jnp.float32),
        ),
        scratch_types=[
            pltpu.VMEM((BATCH,), jnp.int32),
            pltpu.VMEM((BATCH,), jnp.int32),
            pltpu.VMEM((BATCH,), jnp.int32),
            pltpu.VMEM((_B_PER_W,), jnp.int32),
            pltpu.VMEM((8, 8, _W_ROWS), jnp.float32),
            pltpu.VMEM((8, 8, _W_ROWS), jnp.float32),
            pltpu.VMEM((_N_TAIL, N_DIM), jnp.float32),
            pltpu.VMEM((_L, 128), jnp.float32),
            pltpu.VMEM((_L,), jnp.int32),
            pltpu.VMEM((_B_PER_W,), jnp.float32),
            pltpu.SemaphoreType.DMA,
            pltpu.SemaphoreType.DMA,
            pltpu.SemaphoreType.DMA,
            pltpu.SemaphoreType.DMA,
        ],
        compiler_params=pltpu.CompilerParams(use_tc_tiling_on_sc=True,
                                             needs_layout_passes=False),
    )
    bias_out, out_pad = k(idx, vw_t, tail, bias_flat)
    vect_out = lax.slice(out_pad, (0, 0), (BATCH, N_DIM))
    return bias_out, vect_out
